# Initial kernel scaffold; baseline (speedup 1.0000x reference)
import functools

import jax
import jax.numpy as jnp
from jax import lax
from jax.experimental import pallas as pl
from jax.experimental.pallas import tpu as pltpu
from jax.experimental.pallas import tpu_sc as plsc

MAX_LEN = 200
EMBED = 64
NC, NS = 2, 16
NW = NC * NS
CHUNK = 128
LANES = 16


@functools.lru_cache(maxsize=None)
def _make_kernel(n_rows: int):
    rows_per_w = n_rows // NW
    n_chunks = rows_per_w // CHUNK
    assert rows_per_w % CHUNK == 0
    mesh = plsc.VectorSubcoreMesh(core_axis_name="c", subcore_axis_name="s")

    @functools.partial(
        pl.kernel,
        mesh=mesh,
        out_type=jax.ShapeDtypeStruct((n_rows * EMBED,), jnp.float32),
        scratch_types=[
            pltpu.VMEM((CHUNK,), jnp.int32),
            pltpu.VMEM((CHUNK,), jnp.int32),
            pltpu.VMEM((CHUNK,), jnp.float32),
            pltpu.VMEM((CHUNK, EMBED), jnp.float32),
            pltpu.VMEM((CHUNK * EMBED,), jnp.float32),
            pltpu.VMEM((2 * MAX_LEN * EMBED,), jnp.float32),
            pltpu.VMEM((EMBED,), jnp.float32),
            pltpu.SemaphoreType.DMA,
        ],
    )
    def k(seq_hbm, seg_hbm, tok_hbm, base2_hbm, diff_hbm, out_hbm,
          idx_v, segi_v, segf_v, tokbuf, outbuf, base2_v, diff_v, gsem):
        wid = lax.axis_index("s") * NC + lax.axis_index("c")
        w_base = wid * rows_per_w
        pltpu.sync_copy(base2_hbm, base2_v)
        pltpu.sync_copy(diff_hbm, diff_v)
        diff_regs = [diff_v[pl.ds(j * LANES, LANES)] for j in range(EMBED // LANES)]

        def chunk_body(c, carry):
            base = w_base + c * CHUNK
            l0 = lax.rem(base, MAX_LEN)
            pltpu.sync_copy(seq_hbm.at[pl.ds(base, CHUNK)], idx_v)
            pltpu.sync_copy(seg_hbm.at[pl.ds(base, CHUNK)], segi_v)
            pltpu.async_copy(tok_hbm.at[idx_v], tokbuf, gsem).wait()
            for g in range(CHUNK // LANES):
                segf_v[pl.ds(g * LANES, LANES)] = (
                    segi_v[pl.ds(g * LANES, LANES)].astype(jnp.float32))
            for i in range(CHUNK):
                segb = plsc.load_gather(
                    segf_v, [jnp.full((LANES,), i, jnp.int32)])
                prow = (l0 + i) * EMBED
                for j in range(EMBED // LANES):
                    t = tokbuf[i, pl.ds(j * LANES, LANES)]
                    p = base2_v[pl.ds(prow + j * LANES, LANES)]
                    outbuf[pl.ds(i * EMBED + j * LANES, LANES)] = (
                        t + p + segb * diff_regs[j])
            pltpu.sync_copy(outbuf,
                            out_hbm.at[pl.ds(base * EMBED, CHUNK * EMBED)])
            return carry

        lax.fori_loop(0, n_chunks, chunk_body, 0)

    return k


def kernel(seq, seg, token_emb, seg_emb, pos_emb):
    b, l = seq.shape
    n_rows = b * l
    seq_flat = seq.reshape(-1).astype(jnp.int32)
    seg_flat = seg.reshape(-1).astype(jnp.int32)
    base2 = jnp.tile(pos_emb + seg_emb[0], (2, 1)).reshape(-1)
    diff = (seg_emb[1] - seg_emb[0]).reshape(-1)
    out = _make_kernel(n_rows)(seq_flat, seg_flat, token_emb, base2, diff)
    return out.reshape(b, l, EMBED)


# speedup vs baseline: 3.1999x; 3.1999x over previous
import functools

import jax
import jax.numpy as jnp
from jax import lax
from jax.experimental import pallas as pl
from jax.experimental.pallas import tpu as pltpu
from jax.experimental.pallas import tpu_sc as plsc

MAX_LEN = 200
EMBED = 64
NC, NS = 2, 16
NW = NC * NS
CHUNK = 128
LANES = 16


@functools.lru_cache(maxsize=None)
def _make_kernel(n_rows: int):
    rows_per_w = n_rows // NW
    n_chunks = rows_per_w // CHUNK
    assert rows_per_w % CHUNK == 0
    mesh = plsc.VectorSubcoreMesh(core_axis_name="c", subcore_axis_name="s")

    @functools.partial(
        pl.kernel,
        mesh=mesh,
        compiler_params=pltpu.CompilerParams(use_tc_tiling_on_sc=False),
        out_type=jax.ShapeDtypeStruct((n_rows * EMBED,), jnp.float32),
        scratch_types=[
            pltpu.VMEM((CHUNK,), jnp.int32),
            pltpu.VMEM((CHUNK,), jnp.int32),
            pltpu.VMEM((CHUNK,), jnp.float32),
            pltpu.VMEM((CHUNK, EMBED), jnp.float32),
            pltpu.VMEM((CHUNK * EMBED,), jnp.float32),
            pltpu.VMEM((2 * MAX_LEN * EMBED,), jnp.float32),
            pltpu.VMEM((EMBED,), jnp.float32),
            pltpu.SemaphoreType.DMA,
        ],
    )
    def k(seq_hbm, seg_hbm, tok_hbm, base2_hbm, diff_hbm, out_hbm,
          idx_v, segi_v, segf_v, tokbuf, outbuf, base2_v, diff_v, gsem):
        wid = lax.axis_index("s") * NC + lax.axis_index("c")
        w_base = wid * rows_per_w
        pltpu.sync_copy(base2_hbm, base2_v)
        pltpu.sync_copy(diff_hbm, diff_v)
        diff_regs = [diff_v[pl.ds(j * LANES, LANES)] for j in range(EMBED // LANES)]

        def chunk_body(c, carry):
            base = w_base + c * CHUNK
            l0 = lax.rem(base, MAX_LEN)
            pltpu.sync_copy(seq_hbm.at[pl.ds(base, CHUNK)], idx_v)
            pltpu.sync_copy(seg_hbm.at[pl.ds(base, CHUNK)], segi_v)
            pltpu.async_copy(tok_hbm.at[idx_v], tokbuf, gsem).wait()
            for g in range(CHUNK // LANES):
                segf_v[pl.ds(g * LANES, LANES)] = (
                    segi_v[pl.ds(g * LANES, LANES)].astype(jnp.float32))
            for i in range(CHUNK):
                if i % LANES == 0:
                    sv = segf_v[pl.ds(i, LANES)]
                segb = sv[i % LANES]
                prow = (l0 + i) * EMBED
                for j in range(EMBED // LANES):
                    t = tokbuf[i, pl.ds(j * LANES, LANES)]
                    p = base2_v[pl.ds(prow + j * LANES, LANES)]
                    outbuf[pl.ds(i * EMBED + j * LANES, LANES)] = (
                        t + p + segb * diff_regs[j])
            pltpu.sync_copy(outbuf,
                            out_hbm.at[pl.ds(base * EMBED, CHUNK * EMBED)])
            return carry

        lax.fori_loop(0, n_chunks, chunk_body, 0)

    return k


def kernel(seq, seg, token_emb, seg_emb, pos_emb):
    b, l = seq.shape
    n_rows = b * l
    seq_flat = seq.reshape(-1).astype(jnp.int32)
    seg_flat = seg.reshape(-1).astype(jnp.int32)
    base2 = jnp.tile(pos_emb + seg_emb[0], (2, 1)).reshape(-1)
    diff = (seg_emb[1] - seg_emb[0]).reshape(-1)
    out = _make_kernel(n_rows)(seq_flat, seg_flat, token_emb, base2, diff)
    return out.reshape(b, l, EMBED)


# speedup vs baseline: 5.1510x; 1.6097x over previous
import functools

import jax
import jax.numpy as jnp
from jax import lax
from jax.experimental import pallas as pl
from jax.experimental.pallas import tpu as pltpu
from jax.experimental.pallas import tpu_sc as plsc

MAX_LEN = 200
EMBED = 64
NC, NS = 2, 16
NW = NC * NS
CHUNK = 128
LANES = 16


@functools.lru_cache(maxsize=None)
def _make_kernel(n_rows: int):
    rows_per_w = n_rows // NW
    n_chunks = rows_per_w // CHUNK
    assert rows_per_w % CHUNK == 0
    mesh = plsc.VectorSubcoreMesh(core_axis_name="c", subcore_axis_name="s")

    @functools.partial(
        pl.kernel,
        mesh=mesh,
        compiler_params=pltpu.CompilerParams(use_tc_tiling_on_sc=False),
        out_type=jax.ShapeDtypeStruct((n_rows, EMBED), jnp.float32),
        scratch_types=[
            pltpu.VMEM((CHUNK,), jnp.int32),
            pltpu.VMEM((CHUNK,), jnp.int32),
            pltpu.VMEM((CHUNK,), jnp.int32),
            pltpu.VMEM((CHUNK,), jnp.int32),
            pltpu.VMEM((CHUNK, EMBED), jnp.float32),
            pltpu.VMEM_SHARED((4 * MAX_LEN, EMBED), jnp.float32),
            pltpu.SemaphoreType.DMA,
        ],
    )
    def k(seq_hbm, seg_hbm, tok_hbm, com2_hbm, out_hbm,
          idx_v, segi_v, cidx_v, ident_v, outbuf, com2_sh, gsem):
        cid = lax.axis_index("c")
        sid = lax.axis_index("s")
        wid = sid * NC + cid
        w_base = wid * rows_per_w
        iota = lax.broadcasted_iota(jnp.int32, (LANES,), 0)
        for g in range(CHUNK // LANES):
            ident_v[pl.ds(g * LANES, LANES)] = iota + (g * LANES)

        @pl.when(sid == 0)
        def _():
            pltpu.sync_copy(com2_hbm, com2_sh)
        plsc.subcore_barrier()

        def chunk_body(c, carry):
            base = w_base + c * CHUNK
            l0 = lax.rem(base, MAX_LEN)
            pltpu.sync_copy(seq_hbm.at[pl.ds(base, CHUNK)], idx_v)
            pltpu.sync_copy(seg_hbm.at[pl.ds(base, CHUNK)], segi_v)
            gcopy = pltpu.async_copy(tok_hbm.at[idx_v], outbuf, gsem)
            for g in range(CHUNK // LANES):
                sl = pl.ds(g * LANES, LANES)
                cidx_v[sl] = segi_v[sl] * (2 * MAX_LEN) + ident_v[sl] + l0
            gcopy.wait()
            pltpu.sync_copy(com2_sh.at[cidx_v], outbuf, add=True)
            pltpu.sync_copy(outbuf, out_hbm.at[pl.ds(base, CHUNK)])
            return carry

        lax.fori_loop(0, n_chunks, chunk_body, 0)

    return k


def kernel(seq, seg, token_emb, seg_emb, pos_emb):
    b, l = seq.shape
    n_rows = b * l
    seq_flat = seq.reshape(-1).astype(jnp.int32)
    seg_flat = seg.reshape(-1).astype(jnp.int32)
    pos2 = jnp.tile(pos_emb, (2, 1))
    com2 = jnp.concatenate([pos2 + seg_emb[0], pos2 + seg_emb[1]], axis=0)
    out = _make_kernel(n_rows)(seq_flat, seg_flat, token_emb, com2)
    return out.reshape(b, l, EMBED)


# speedup vs baseline: 6.5298x; 1.2677x over previous
import functools

import jax
import jax.numpy as jnp
from jax import lax
from jax.experimental import pallas as pl
from jax.experimental.pallas import tpu as pltpu
from jax.experimental.pallas import tpu_sc as plsc

MAX_LEN = 200
EMBED = 64
NC, NS = 2, 16
NW = NC * NS
CHUNK = 128
LANES = 16


@functools.lru_cache(maxsize=None)
def _make_kernel(n_rows: int):
    rows_per_w = n_rows // NW
    n_chunks = rows_per_w // CHUNK
    assert rows_per_w % CHUNK == 0 and n_chunks % 2 == 0
    mesh = plsc.VectorSubcoreMesh(core_axis_name="c", subcore_axis_name="s")

    buf2 = lambda *shape: pltpu.VMEM(shape, jnp.int32)

    @functools.partial(
        pl.kernel,
        mesh=mesh,
        compiler_params=pltpu.CompilerParams(use_tc_tiling_on_sc=False),
        out_type=jax.ShapeDtypeStruct((n_rows, EMBED), jnp.float32),
        scratch_types=[
            [buf2(CHUNK), buf2(CHUNK)],
            [buf2(CHUNK), buf2(CHUNK)],
            pltpu.VMEM((CHUNK,), jnp.int32),
            pltpu.VMEM((CHUNK,), jnp.int32),
            [pltpu.VMEM((CHUNK, EMBED), jnp.float32),
             pltpu.VMEM((CHUNK, EMBED), jnp.float32)],
            pltpu.VMEM_SHARED((4 * MAX_LEN, EMBED), jnp.float32),
            [pltpu.SemaphoreType.DMA, pltpu.SemaphoreType.DMA],
            [pltpu.SemaphoreType.DMA, pltpu.SemaphoreType.DMA],
        ],
    )
    def k(seq_hbm, seg_hbm, tok_hbm, com2_hbm, out_hbm,
          idx_v, segi_v, cidx_v, ident_v, outbuf, com2_sh, gsem, osem):
        cid = lax.axis_index("c")
        sid = lax.axis_index("s")
        wid = sid * NC + cid
        w_base = wid * rows_per_w
        iota = lax.broadcasted_iota(jnp.int32, (LANES,), 0)
        for g in range(CHUNK // LANES):
            ident_v[pl.ds(g * LANES, LANES)] = iota + (g * LANES)

        @pl.when(sid == 0)
        def _():
            pltpu.sync_copy(com2_hbm, com2_sh)
        plsc.subcore_barrier()

        def prefetch(c, p):
            base = w_base + c * CHUNK
            pltpu.sync_copy(seq_hbm.at[pl.ds(base, CHUNK)], idx_v[p])
            pltpu.sync_copy(seg_hbm.at[pl.ds(base, CHUNK)], segi_v[p])

        def issue_gather(p):
            return pltpu.async_copy(tok_hbm.at[idx_v[p]], outbuf[p], gsem[p])

        def halfstep(c, p, q):
            base = w_base + c * CHUNK
            l0 = lax.rem(base, MAX_LEN)

            @pl.when(c > 0)
            def _():
                pltpu.make_async_copy(outbuf[q], out_hbm.at[pl.ds(0, CHUNK)],
                                      osem[q]).wait()

            @pl.when(c + 1 < n_chunks)
            def _():
                issue_gather(q)

            pltpu.make_async_copy(tok_hbm.at[idx_v[p]], outbuf[p],
                                  gsem[p]).wait()
            for g in range(CHUNK // LANES):
                sl = pl.ds(g * LANES, LANES)
                cidx_v[sl] = segi_v[p][sl] * (2 * MAX_LEN) + ident_v[sl] + l0
            pltpu.sync_copy(com2_sh.at[cidx_v], outbuf[p], add=True)
            pltpu.async_copy(outbuf[p], out_hbm.at[pl.ds(base, CHUNK)],
                             osem[p])

            @pl.when(c + 2 < n_chunks)
            def _():
                prefetch(c + 2, p)

        prefetch(0, 0)
        prefetch(1, 1)
        issue_gather(0)

        def body(t, carry):
            halfstep(2 * t, 0, 1)
            halfstep(2 * t + 1, 1, 0)
            return carry

        lax.fori_loop(0, n_chunks // 2, body, 0)
        pltpu.make_async_copy(outbuf[1], out_hbm.at[pl.ds(0, CHUNK)],
                              osem[1]).wait()

    return k


def kernel(seq, seg, token_emb, seg_emb, pos_emb):
    b, l = seq.shape
    n_rows = b * l
    seq_flat = seq.reshape(-1).astype(jnp.int32)
    seg_flat = seg.reshape(-1).astype(jnp.int32)
    pos2 = jnp.tile(pos_emb, (2, 1))
    com2 = jnp.concatenate([pos2 + seg_emb[0], pos2 + seg_emb[1]], axis=0)
    out = _make_kernel(n_rows)(seq_flat, seg_flat, token_emb, com2)
    return out.reshape(b, l, EMBED)


# speedup vs baseline: 6.9440x; 1.0634x over previous
import functools

import jax
import jax.numpy as jnp
from jax import lax
from jax.experimental import pallas as pl
from jax.experimental.pallas import tpu as pltpu
from jax.experimental.pallas import tpu_sc as plsc

MAX_LEN = 200
EMBED = 64
NC, NS = 2, 16
NW = NC * NS
CHUNK = 128
LANES = 16


@functools.lru_cache(maxsize=None)
def _make_kernel(n_rows: int):
    rows_per_w = n_rows // NW
    n_chunks = rows_per_w // CHUNK
    assert rows_per_w % CHUNK == 0 and n_chunks % 2 == 0
    mesh = plsc.VectorSubcoreMesh(core_axis_name="c", subcore_axis_name="s")

    buf2 = lambda *shape: pltpu.VMEM(shape, jnp.int32)

    @functools.partial(
        pl.kernel,
        mesh=mesh,
        compiler_params=pltpu.CompilerParams(use_tc_tiling_on_sc=False),
        out_type=jax.ShapeDtypeStruct((n_rows, EMBED), jnp.float32),
        scratch_types=[
            [buf2(CHUNK), buf2(CHUNK)],
            [buf2(CHUNK), buf2(CHUNK)],
            pltpu.VMEM((CHUNK,), jnp.int32),
            pltpu.VMEM((CHUNK,), jnp.int32),
            [pltpu.VMEM((CHUNK, EMBED), jnp.float32),
             pltpu.VMEM((CHUNK, EMBED), jnp.float32)],
            pltpu.VMEM_SHARED((4 * MAX_LEN, EMBED), jnp.float32),
            [pltpu.SemaphoreType.DMA, pltpu.SemaphoreType.DMA],
            [pltpu.SemaphoreType.DMA, pltpu.SemaphoreType.DMA],
        ],
    )
    def k(seq_hbm, seg_hbm, tok_hbm, com2_hbm, out_hbm,
          idx_v, segi_v, cidx_v, ident_v, outbuf, com2_sh, gsem, osem):
        cid = lax.axis_index("c")
        sid = lax.axis_index("s")
        wid = sid * NC + cid
        w_base = wid * rows_per_w
        iota = lax.broadcasted_iota(jnp.int32, (LANES,), 0)
        for g in range(CHUNK // LANES):
            ident_v[pl.ds(g * LANES, LANES)] = iota + (g * LANES)

        @pl.when(sid == 0)
        def _():
            pltpu.sync_copy(com2_hbm, com2_sh)
        plsc.subcore_barrier()

        def prefetch(c, p):
            base = w_base + c * CHUNK
            pltpu.sync_copy(seq_hbm.at[pl.ds(base, CHUNK)], idx_v[p])
            pltpu.sync_copy(seg_hbm.at[pl.ds(base, CHUNK)], segi_v[p])

        def issue_gather(p):
            return pltpu.async_copy(tok_hbm.at[idx_v[p]], outbuf[p], gsem[p])

        def halfstep(c, p, q):
            base = w_base + c * CHUNK
            l0 = lax.rem(base, MAX_LEN)

            @pl.when(c > 0)
            def _():
                pltpu.make_async_copy(outbuf[q], out_hbm.at[pl.ds(0, CHUNK)],
                                      osem[q]).wait()

            @pl.when(c + 1 < n_chunks)
            def _():
                issue_gather(q)

            pltpu.make_async_copy(tok_hbm.at[idx_v[p]], outbuf[p],
                                  gsem[p]).wait()
            for g in range(CHUNK // LANES):
                sl = pl.ds(g * LANES, LANES)
                cidx_v[sl] = segi_v[p][sl] * (2 * MAX_LEN) + ident_v[sl] + l0
            pltpu.sync_copy(com2_sh.at[cidx_v], outbuf[p], add=True)
            pltpu.async_copy(outbuf[p], out_hbm.at[pl.ds(base, CHUNK)],
                             osem[p])

            @pl.when(c + 2 < n_chunks)
            def _():
                prefetch(c + 2, p)

        prefetch(0, 0)
        prefetch(1, 1)
        issue_gather(0)

        def body(t, carry):
            halfstep(2 * t, 0, 1)
            halfstep(2 * t + 1, 1, 0)
            return carry

        lax.fori_loop(0, n_chunks // 2, body, 0)
        pltpu.make_async_copy(outbuf[1], out_hbm.at[pl.ds(0, CHUNK)],
                              osem[1]).wait()

    return k


N_SLICES = 4


def kernel(seq, seg, token_emb, seg_emb, pos_emb):
    b, l = seq.shape
    seq_flat = seq.reshape(-1).astype(jnp.int32)
    seg_flat = seg.reshape(-1).astype(jnp.int32)
    pos2 = jnp.tile(pos_emb, (2, 1))
    com2 = jnp.concatenate([pos2 + seg_emb[0], pos2 + seg_emb[1]], axis=0)
    bs = b // N_SLICES
    rows = bs * l
    k = _make_kernel(rows)
    outs = [
        k(lax.dynamic_slice_in_dim(seq_flat, i * rows, rows),
          lax.dynamic_slice_in_dim(seg_flat, i * rows, rows),
          token_emb, com2).reshape(bs, l, EMBED)
        for i in range(N_SLICES)
    ]
    return jnp.concatenate(outs, axis=0)
